# baseline (speedup 1.0000x reference)
import jax
import jax.numpy as jnp
from jax.experimental import pallas as pl
from jax.experimental.pallas import tpu as pltpu

B, N, D = 32, 128, 128
G = 4
SIGMA = 2.5


def _body(c_o, ct_o, x_o, c_m, ct_m, x_m,
          w1, b1, w2, b2, wt1, bt1, wt2, bt2, out):
    w1v = w1[...].astype(jnp.bfloat16)
    w2v = w2[...].astype(jnp.bfloat16)
    b1v = b1[...]
    b2v = b2[...]

    ii = jax.lax.broadcasted_iota(jnp.int32, (N, N), 0)
    jj = jax.lax.broadcasted_iota(jnp.int32, (N, N), 1)
    eyef = jnp.where(ii == jj, jnp.float32(1.0), jnp.float32(0.0))
    maskf = 1.0 - eyef

    def pooled(c, ct, x):
        d2 = (c[:, 0:1] - ct[0:1, :]) ** 2
        d2 += (c[:, 1:2] - ct[1:2, :]) ** 2
        d2 += (c[:, 2:3] - ct[2:3, :]) ** 2
        dist = jnp.sqrt(d2 + 1e-12)
        A = jnp.exp(dist * jnp.float32(-1.0 / SIGMA))
        A = A * maskf + eyef

        dinv_r = 1.0 / jnp.sqrt(jnp.sum(A, axis=1, keepdims=True))
        dinv_c = 1.0 / jnp.sqrt(jnp.sum(A, axis=0, keepdims=True))
        an = ((A * dinv_r) * dinv_c).astype(jnp.bfloat16)

        h = x
        for wv, bv in ((w1v, b1v), (w2v, b2v)):
            z = jnp.dot(an, h.astype(jnp.bfloat16),
                        preferred_element_type=jnp.float32)
            z = jnp.dot(z.astype(jnp.bfloat16), wv,
                        preferred_element_type=jnp.float32)
            h = jnp.maximum(z + bv, 0.0)
        return jnp.max(h, axis=0, keepdims=True)

    po = jnp.concatenate(
        [pooled(c_o[g], ct_o[g], x_o[g]) for g in range(G)], axis=0)
    pm = jnp.concatenate(
        [pooled(c_m[g], ct_m[g], x_m[g]) for g in range(G)], axis=0)

    t = jnp.dot(po.astype(jnp.bfloat16), wt1[0:D, :].astype(jnp.bfloat16),
                preferred_element_type=jnp.float32)
    t += jnp.dot(pm.astype(jnp.bfloat16), wt1[D:2 * D, :].astype(jnp.bfloat16),
                 preferred_element_type=jnp.float32)
    t = jnp.maximum(t + bt1[...], 0.0)
    tb = t.astype(jnp.bfloat16).astype(jnp.float32)
    wb = wt2[...].astype(jnp.bfloat16).astype(jnp.float32)
    s = jnp.sum(tb * wb, axis=1, keepdims=True) + bt2[0, 0]
    out[...] = (jnp.zeros((G, D), jnp.float32) + s)[None]


def kernel(coords_orig, feats_orig, coords_mut, feats_mut,
           W1, b1, W2, b2, Wt1, bt1, Wt2, bt2):
    ct_o = jnp.swapaxes(coords_orig, 1, 2)
    ct_m = jnp.swapaxes(coords_mut, 1, 2)

    per_chunk = lambda i: (i, 0, 0)
    const2 = lambda i: (0, 0)

    return pl.pallas_call(
        _body,
        grid=(B // G,),
        in_specs=[
            pl.BlockSpec((G, N, 3), per_chunk),
            pl.BlockSpec((G, 3, N), per_chunk),
            pl.BlockSpec((G, N, D), per_chunk),
            pl.BlockSpec((G, N, 3), per_chunk),
            pl.BlockSpec((G, 3, N), per_chunk),
            pl.BlockSpec((G, N, D), per_chunk),
            pl.BlockSpec((D, D), const2),
            pl.BlockSpec((1, D), const2),
            pl.BlockSpec((D, D), const2),
            pl.BlockSpec((1, D), const2),
            pl.BlockSpec((2 * D, D), const2),
            pl.BlockSpec((1, D), const2),
            pl.BlockSpec((1, D), const2),
            pl.BlockSpec((1, 1), const2),
        ],
        out_specs=pl.BlockSpec((1, G, D), lambda i: (i, 0, 0)),
        out_shape=jax.ShapeDtypeStruct((B // G, G, D), jnp.float32),
        compiler_params=pltpu.CompilerParams(
            dimension_semantics=("parallel",)),
    )(coords_orig, ct_o, feats_orig, coords_mut, ct_m, feats_mut,
      W1, b1.reshape(1, D), W2, b2.reshape(1, D),
      Wt1, bt1.reshape(1, D), Wt2.reshape(1, D),
      bt2.reshape(1, 1)).reshape(B, D)[:, :1]


# speedup vs baseline: 1.0900x; 1.0900x over previous
import jax
import jax.numpy as jnp
from jax.experimental import pallas as pl
from jax.experimental.pallas import tpu as pltpu

B, N, D = 32, 128, 128
G = 16
SIGMA = 2.5


def _body(c_o, ct_o, x_o, c_m, ct_m, x_m,
          w1, b1, w2, b2, wt1, bt1, wt2, bt2, out):
    w1v = w1[...].astype(jnp.bfloat16)
    w2v = w2[...].astype(jnp.bfloat16)
    b1v = b1[...]
    b2v = b2[...]

    ii = jax.lax.broadcasted_iota(jnp.int32, (N, N), 0)
    jj = jax.lax.broadcasted_iota(jnp.int32, (N, N), 1)
    eyef = jnp.where(ii == jj, jnp.float32(1.0), jnp.float32(0.0))
    maskf = 1.0 - eyef

    def pooled(c, ct, x):
        d2 = (c[:, 0:1] - ct[0:1, :]) ** 2
        d2 += (c[:, 1:2] - ct[1:2, :]) ** 2
        d2 += (c[:, 2:3] - ct[2:3, :]) ** 2
        dist = jnp.sqrt(d2 + 1e-12)
        A = jnp.exp(dist * jnp.float32(-1.0 / SIGMA))
        A = A * maskf + eyef

        dinv_r = 1.0 / jnp.sqrt(jnp.sum(A, axis=1, keepdims=True))
        dinv_c = 1.0 / jnp.sqrt(jnp.sum(A, axis=0, keepdims=True))
        an = ((A * dinv_r) * dinv_c).astype(jnp.bfloat16)

        h = x
        for wv, bv in ((w1v, b1v), (w2v, b2v)):
            z = jnp.dot(an, h.astype(jnp.bfloat16),
                        preferred_element_type=jnp.float32)
            z = jnp.dot(z.astype(jnp.bfloat16), wv,
                        preferred_element_type=jnp.float32)
            h = jnp.maximum(z + bv, 0.0)
        return jnp.max(h, axis=0, keepdims=True)

    po = jnp.concatenate(
        [pooled(c_o[g], ct_o[g], x_o[g]) for g in range(G)], axis=0)
    pm = jnp.concatenate(
        [pooled(c_m[g], ct_m[g], x_m[g]) for g in range(G)], axis=0)

    t = jnp.dot(po.astype(jnp.bfloat16), wt1[0:D, :].astype(jnp.bfloat16),
                preferred_element_type=jnp.float32)
    t += jnp.dot(pm.astype(jnp.bfloat16), wt1[D:2 * D, :].astype(jnp.bfloat16),
                 preferred_element_type=jnp.float32)
    t = jnp.maximum(t + bt1[...], 0.0)
    tb = t.astype(jnp.bfloat16).astype(jnp.float32)
    wb = wt2[...].astype(jnp.bfloat16).astype(jnp.float32)
    s = jnp.sum(tb * wb, axis=1, keepdims=True) + bt2[0, 0]
    out[...] = (jnp.zeros((G, D), jnp.float32) + s)[None]


def kernel(coords_orig, feats_orig, coords_mut, feats_mut,
           W1, b1, W2, b2, Wt1, bt1, Wt2, bt2):
    ct_o = jnp.swapaxes(coords_orig, 1, 2)
    ct_m = jnp.swapaxes(coords_mut, 1, 2)

    per_chunk = lambda i: (i, 0, 0)
    const2 = lambda i: (0, 0)

    return pl.pallas_call(
        _body,
        grid=(B // G,),
        in_specs=[
            pl.BlockSpec((G, N, 3), per_chunk),
            pl.BlockSpec((G, 3, N), per_chunk),
            pl.BlockSpec((G, N, D), per_chunk),
            pl.BlockSpec((G, N, 3), per_chunk),
            pl.BlockSpec((G, 3, N), per_chunk),
            pl.BlockSpec((G, N, D), per_chunk),
            pl.BlockSpec((D, D), const2),
            pl.BlockSpec((1, D), const2),
            pl.BlockSpec((D, D), const2),
            pl.BlockSpec((1, D), const2),
            pl.BlockSpec((2 * D, D), const2),
            pl.BlockSpec((1, D), const2),
            pl.BlockSpec((1, D), const2),
            pl.BlockSpec((1, 1), const2),
        ],
        out_specs=pl.BlockSpec((1, G, D), lambda i: (i, 0, 0)),
        out_shape=jax.ShapeDtypeStruct((B // G, G, D), jnp.float32),
        compiler_params=pltpu.CompilerParams(
            dimension_semantics=("parallel",)),
    )(coords_orig, ct_o, feats_orig, coords_mut, ct_m, feats_mut,
      W1, b1.reshape(1, D), W2, b2.reshape(1, D),
      Wt1, bt1.reshape(1, D), Wt2.reshape(1, D),
      bt2.reshape(1, 1)).reshape(B, D)[:, :1]
